# baseline (speedup 1.0000x reference)
import functools

import jax
import jax.numpy as jnp
from jax import lax
from jax.experimental import pallas as pl
from jax.experimental.pallas import tpu as pltpu
from jax.experimental.pallas import tpu_sc as plsc


@functools.cache
def _make_gather(V, D, B, S):
    info = plsc.get_sparse_core_info()
    NC, NS = info.num_cores, info.num_subcores
    NW = NC * NS
    N = B * S
    assert N % NW == 0
    n_per_w = N // NW
    assert n_per_w % 8 == 0 and S % n_per_w == 0
    wpb = S // n_per_w
    C = 16
    assert n_per_w % C == 0 and C % 8 == 0
    nch = n_per_w // C
    mesh = plsc.VectorSubcoreMesh(core_axis_name="c", subcore_axis_name="s")

    @functools.partial(
        pl.kernel,
        mesh=mesh,
        out_type=jax.ShapeDtypeStruct((B, S, D), jnp.float32),
        scratch_types=[
            pltpu.VMEM((n_per_w,), jnp.int32),
            pltpu.VMEM((C, D), jnp.float32),
            pltpu.VMEM((C, D), jnp.float32),
            pltpu.SemaphoreType.DMA,
            pltpu.SemaphoreType.DMA,
            pltpu.SemaphoreType.DMA,
            pltpu.SemaphoreType.DMA,
        ],
    )
    def gather_kernel(table_hbm, idx_hbm, out_hbm, idx_v, rows0, rows1,
                      gsem0, gsem1, wsem0, wsem1):
        wid = lax.axis_index("s") * NC + lax.axis_index("c")
        b = wid // wpb
        s0 = (wid % wpb) * n_per_w
        pltpu.sync_copy(idx_hbm.at[b, pl.ds(s0, n_per_w)], idx_v)

        bufs = (rows0, rows1)
        gsems = (gsem0, gsem1)
        wsems = (wsem0, wsem1)
        gathers = [None, None]
        writes = [None, None]
        for c in range(nch):
            i = c & 1
            if writes[i] is not None:
                writes[i].wait()
            gathers[i] = pltpu.async_copy(
                table_hbm.at[idx_v.at[pl.ds(c * C, C)]], bufs[i], gsems[i])
            if c >= 1:
                j = (c - 1) & 1
                gathers[j].wait()
                writes[j] = pltpu.async_copy(
                    bufs[j], out_hbm.at[b, pl.ds(s0 + (c - 1) * C, C)],
                    wsems[j])
        last = (nch - 1) & 1
        gathers[last].wait()
        writes[last] = pltpu.async_copy(
            bufs[last], out_hbm.at[b, pl.ds(s0 + (nch - 1) * C, C)],
            wsems[last])
        writes[1 - last].wait()
        writes[last].wait()

    return gather_kernel


def kernel(input_ids, embed_weight):
    B, S = input_ids.shape
    V, D = embed_weight.shape
    return _make_gather(V, D, B, S)(embed_weight, input_ids)


# speedup vs baseline: 1.0397x; 1.0397x over previous
import functools

import jax
import jax.numpy as jnp
from jax import lax
from jax.experimental import pallas as pl
from jax.experimental.pallas import tpu as pltpu
from jax.experimental.pallas import tpu_sc as plsc


@functools.cache
def _make_gather(V, D, B, S):
    info = plsc.get_sparse_core_info()
    NC, NS = info.num_cores, info.num_subcores
    NW = NC * NS
    N = B * S
    assert N % NW == 0
    n_per_w = N // NW
    assert n_per_w % 8 == 0 and S % n_per_w == 0
    wpb = S // n_per_w
    C = 32
    assert n_per_w % C == 0 and C % 8 == 0
    nch = n_per_w // C
    mesh = plsc.VectorSubcoreMesh(core_axis_name="c", subcore_axis_name="s")

    @functools.partial(
        pl.kernel,
        mesh=mesh,
        out_type=jax.ShapeDtypeStruct((B, S, D), jnp.float32),
        scratch_types=[
            pltpu.VMEM((n_per_w,), jnp.int32),
            pltpu.VMEM((C, D), jnp.float32),
            pltpu.VMEM((C, D), jnp.float32),
            pltpu.SemaphoreType.DMA,
            pltpu.SemaphoreType.DMA,
            pltpu.SemaphoreType.DMA,
            pltpu.SemaphoreType.DMA,
        ],
    )
    def gather_kernel(table_hbm, idx_hbm, out_hbm, idx_v, rows0, rows1,
                      gsem0, gsem1, wsem0, wsem1):
        wid = lax.axis_index("s") * NC + lax.axis_index("c")
        b = wid // wpb
        s0 = (wid % wpb) * n_per_w
        pltpu.sync_copy(idx_hbm.at[b, pl.ds(s0, n_per_w)], idx_v)

        bufs = (rows0, rows1)
        gsems = (gsem0, gsem1)
        wsems = (wsem0, wsem1)
        gathers = [None, None]
        writes = [None, None]
        for c in range(nch):
            i = c & 1
            if writes[i] is not None:
                writes[i].wait()
            gathers[i] = pltpu.async_copy(
                table_hbm.at[idx_v.at[pl.ds(c * C, C)]], bufs[i], gsems[i])
            if c >= 1:
                j = (c - 1) & 1
                gathers[j].wait()
                writes[j] = pltpu.async_copy(
                    bufs[j], out_hbm.at[b, pl.ds(s0 + (c - 1) * C, C)],
                    wsems[j])
        last = (nch - 1) & 1
        gathers[last].wait()
        writes[last] = pltpu.async_copy(
            bufs[last], out_hbm.at[b, pl.ds(s0 + (nch - 1) * C, C)],
            wsems[last])
        writes[1 - last].wait()
        writes[last].wait()

    return gather_kernel


def kernel(input_ids, embed_weight):
    B, S = input_ids.shape
    V, D = embed_weight.shape
    return _make_gather(V, D, B, S)(embed_weight, input_ids)
